# baseline (speedup 1.0000x reference)
import functools

import jax
import jax.numpy as jnp
from jax import lax
from jax.experimental import pallas as pl
from jax.experimental.pallas import tpu as pltpu
from jax.experimental.pallas import tpu_sc as plsc

NC = 2
NS = 16
NW = NC * NS

N = 10000
E = 320000
D = 128
T = 32768
NREL = 100

EPW = E // NW
KB = 40
NB = EPW // KB
NCH = 10
CNB = NB // NCH
NPIPE = 5
S_LO = 624
S_HI = 640

TPW = T // NW
TKB = 64
TNB = TPW // TKB

_HIGH = lax.Precision.HIGHEST

_mesh = plsc.VectorSubcoreMesh(core_axis_name="c", subcore_axis_name="s")


def _stripe_copy(sid, refs_fn):
    @pl.when(sid < NS - 1)
    def _():
        src, dst = refs_fn(pl.ds(sid * S_LO, S_LO))
        pltpu.sync_copy(src, dst)

    @pl.when(sid == NS - 1)
    def _():
        src, dst = refs_fn(pl.ds((NS - 1) * S_LO, S_HI))
        pltpu.sync_copy(src, dst)


def _sc_degree(dst3, ones, zeros16):
    @functools.partial(
        pl.kernel,
        out_type=jax.ShapeDtypeStruct((NC, N, D), jnp.float32),
        mesh=_mesh,
        scratch_types=[
            pltpu.VMEM((NB, KB), jnp.int32),
            pltpu.VMEM((KB, D), jnp.float32),
            pltpu.VMEM_SHARED((N, D), jnp.float32),
            pltpu.SemaphoreType.DMA,
        ],
    )
    def k(dst_hbm, ones_hbm, zero_hbm, out_hbm, idx_v, ones_v, acc_sh, sem):
        cid = lax.axis_index("c")
        sid = lax.axis_index("s")
        wid = sid * NC + cid
        _stripe_copy(sid, lambda s: (zero_hbm.at[s], acc_sh.at[s]))
        pltpu.sync_copy(ones_hbm, ones_v)
        pltpu.sync_copy(dst_hbm.at[wid], idx_v)
        plsc.subcore_barrier()

        @pl.loop(0, NB)
        def _(j):
            pltpu.async_copy(ones_v, acc_sh.at[idx_v.at[j]], sem, add=True)

        @pl.loop(0, NB)
        def _(j):
            pltpu.make_async_copy(ones_v, acc_sh.at[idx_v.at[j]], sem).wait()

        plsc.subcore_barrier()
        _stripe_copy(sid, lambda s: (acc_sh.at[s], out_hbm.at[cid].at[s]))

    return k(dst3, ones, zeros16)


def _sc_edge_scatter(g, src3, dst3, zeros):
    @functools.partial(
        pl.kernel,
        out_type=jax.ShapeDtypeStruct((NC, N, D), jnp.float32),
        mesh=_mesh,
        scratch_types=[
            pltpu.VMEM((CNB, KB), jnp.int32),
            pltpu.VMEM((CNB, KB), jnp.int32),
        ] + [pltpu.VMEM((KB, D), jnp.float32)] * NPIPE + [
            pltpu.VMEM_SHARED((N, D), jnp.float32),
        ] + [pltpu.SemaphoreType.DMA] * (2 * NPIPE),
    )
    def k(g_hbm, src_hbm, dst_hbm, zero_hbm, out_hbm, src_v, dst_v, *rest):
        rows = rest[:NPIPE]
        acc_sh = rest[NPIPE]
        gsem = rest[NPIPE + 1:2 * NPIPE + 1]
        ssem = rest[2 * NPIPE + 1:]
        cid = lax.axis_index("c")
        sid = lax.axis_index("s")
        wid = sid * NC + cid
        _stripe_copy(sid, lambda s: (zero_hbm.at[s], acc_sh.at[s]))
        plsc.subcore_barrier()

        for c in range(NCH):
            pltpu.sync_copy(src_hbm.at[wid].at[c], src_v)
            pltpu.sync_copy(dst_hbm.at[wid].at[c], dst_v)
            for b in range(NPIPE - 1):
                pltpu.async_copy(g_hbm.at[src_v.at[b]], rows[b], gsem[b])

            @pl.loop(0, CNB // NPIPE)
            def _(i):
                for off in range(NPIPE):
                    j = NPIPE * i + off
                    nb = (off + NPIPE - 1) % NPIPE
                    pltpu.make_async_copy(
                        g_hbm.at[src_v.at[j]], rows[off], gsem[off]).wait()
                    pltpu.async_copy(rows[off], acc_sh.at[dst_v.at[j]],
                                     ssem[off], add=True)

                    @pl.when(j + NPIPE - 1 < CNB)
                    def _():
                        @pl.when(j >= 1)
                        def _():
                            pltpu.make_async_copy(
                                rows[nb], acc_sh.at[dst_v.at[j - 1]],
                                ssem[nb]).wait()

                        pltpu.async_copy(
                            g_hbm.at[src_v.at[j + NPIPE - 1]], rows[nb],
                            gsem[nb])

            for b in range(NPIPE):
                pltpu.make_async_copy(
                    rows[b], acc_sh.at[dst_v.at[CNB - NPIPE + b]],
                    ssem[b]).wait()

        plsc.subcore_barrier()
        _stripe_copy(sid, lambda s: (acc_sh.at[s], out_hbm.at[cid].at[s]))

    return k(g, src3, dst3, zeros)


def _sc_triple_gather(h, rel_emb, idx3):
    row_t = jax.ShapeDtypeStruct((T, D), jnp.float32)

    @functools.partial(
        pl.kernel,
        out_type=(row_t, row_t, row_t),
        mesh=_mesh,
        scratch_types=[
            pltpu.VMEM((3 * TNB, TKB), jnp.int32),
        ] + [pltpu.VMEM((3, TKB, D), jnp.float32)] * 3
          + [pltpu.SemaphoreType.DMA] * 6,
    )
    def k(h_hbm, rel_hbm, idx_hbm, oh_hbm, orel_hbm, ot_hbm, idx_v, *rest):
        bufs = rest[:3]
        gsem = rest[3:6]
        wsem = rest[6:]
        cid = lax.axis_index("c")
        sid = lax.axis_index("s")
        wid = sid * NC + cid
        base = wid * TPW
        pltpu.sync_copy(idx_hbm.at[wid], idx_v)

        def fire(b, j):
            pltpu.async_copy(h_hbm.at[idx_v.at[j]], bufs[b].at[0], gsem[b])
            pltpu.async_copy(rel_hbm.at[idx_v.at[TNB + j]], bufs[b].at[1],
                             gsem[b])
            pltpu.async_copy(h_hbm.at[idx_v.at[2 * TNB + j]], bufs[b].at[2],
                             gsem[b])

        def wait_gathers(b, j):
            pltpu.make_async_copy(h_hbm.at[idx_v.at[j]], bufs[b].at[0],
                                  gsem[b]).wait()
            pltpu.make_async_copy(rel_hbm.at[idx_v.at[TNB + j]],
                                  bufs[b].at[1], gsem[b]).wait()
            pltpu.make_async_copy(h_hbm.at[idx_v.at[2 * TNB + j]],
                                  bufs[b].at[2], gsem[b]).wait()

        def fire_writes(b, j):
            dst = pl.ds(base + j * TKB, TKB)
            pltpu.async_copy(bufs[b].at[0], oh_hbm.at[dst], wsem[b])
            pltpu.async_copy(bufs[b].at[1], orel_hbm.at[dst], wsem[b])
            pltpu.async_copy(bufs[b].at[2], ot_hbm.at[dst], wsem[b])

        def wait_writes(b, j):
            dst = pl.ds(base + j * TKB, TKB)
            pltpu.make_async_copy(bufs[b].at[0], oh_hbm.at[dst],
                                  wsem[b]).wait()
            pltpu.make_async_copy(bufs[b].at[1], orel_hbm.at[dst],
                                  wsem[b]).wait()
            pltpu.make_async_copy(bufs[b].at[2], ot_hbm.at[dst],
                                  wsem[b]).wait()

        fire(0, 0)
        fire(1, 1)
        for j in range(TNB):
            b = j % 3
            wait_gathers(b, j)
            fire_writes(b, j)
            if j + 2 < TNB:
                nb = (j + 2) % 3
                if j >= 1:
                    wait_writes(nb, j - 1)
                fire(nb, j + 2)
        for j in range(TNB - 3, TNB):
            wait_writes(j % 3, j)

    return k(h, rel_emb, idx3)


def _dinv_from(degp_a, degp_b):
    return lax.rsqrt(degp_a[:, 0:1] + degp_b[:, 0:1] + 1.0)


def _tc_g1(x, W1, degp):
    def body(x_ref, w_ref, deg_ref, o_ref):
        dinv = _dinv_from(deg_ref[0], deg_ref[1])
        h1 = jnp.dot(x_ref[...], w_ref[...],
                     preferred_element_type=jnp.float32, precision=_HIGH)
        o_ref[...] = h1 * dinv

    return pl.pallas_call(
        body, out_shape=jax.ShapeDtypeStruct((N, D), jnp.float32),
    )(x, W1, degp)


def _tc_mid(accp, g1, degp, b1, W2):
    def body(acc_ref, g_ref, deg_ref, b_ref, w_ref, o_ref):
        dinv = _dinv_from(deg_ref[0], deg_ref[1])
        s = acc_ref[0] + acc_ref[1] + g_ref[...]
        a1 = jnp.maximum(s * dinv + b_ref[...], 0.0)
        h2 = jnp.dot(a1, w_ref[...],
                     preferred_element_type=jnp.float32, precision=_HIGH)
        o_ref[...] = h2 * dinv

    return pl.pallas_call(
        body, out_shape=jax.ShapeDtypeStruct((N, D), jnp.float32),
    )(accp, g1, degp, b1, W2)


def _tc_post2(accp, g2, degp, b2):
    def body(acc_ref, g_ref, deg_ref, b_ref, o_ref):
        dinv = _dinv_from(deg_ref[0], deg_ref[1])
        s = acc_ref[0] + acc_ref[1] + g_ref[...]
        o_ref[...] = jnp.maximum(s * dinv + b_ref[...], 0.0)

    return pl.pallas_call(
        body, out_shape=jax.ShapeDtypeStruct((N, D), jnp.float32),
    )(accp, g2, degp, b2)


_TB = 4096


def _tc_mlp(hh, hr, ht, Wm1, bm1, Wm2p, bm2p):
    def body(hh_ref, hr_ref, ht_ref, w1_ref, b1_ref, w2_ref, b2_ref, o_ref):
        t = hh_ref[...] + hr_ref[...] + ht_ref[...]
        q = jnp.maximum(
            jnp.dot(t, w1_ref[...], preferred_element_type=jnp.float32,
                    precision=_HIGH) + b1_ref[...], 0.0)
        o_ref[...] = jnp.dot(q, w2_ref[...],
                             preferred_element_type=jnp.float32,
                             precision=_HIGH) + b2_ref[...]

    row_spec = pl.BlockSpec((_TB, D), lambda i: (i, 0))
    full = pl.BlockSpec((D, D), lambda i: (0, 0))
    vec = pl.BlockSpec((1, D), lambda i: (0, 0))
    return pl.pallas_call(
        body,
        grid=(T // _TB,),
        in_specs=[row_spec, row_spec, row_spec, full, vec, full, vec],
        out_specs=row_spec,
        out_shape=jax.ShapeDtypeStruct((T, D), jnp.float32),
    )(hh, hr, ht, Wm1, bm1, Wm2p, bm2p)


def kernel(x, edge_index, head_idx, tail_idx, rel_idx, W1, b1, W2, b2,
           rel_emb, Wm1, bm1, Wm2, bm2):
    src3 = edge_index[0].reshape(NW, NB, KB)
    dst3 = edge_index[1].reshape(NW, NB, KB)
    src4 = src3.reshape(NW, NCH, CNB, KB)
    dst4 = dst3.reshape(NW, NCH, CNB, KB)

    ones = jnp.ones((KB, D), jnp.float32)
    zeros = jnp.zeros((N, D), jnp.float32)

    hh3 = head_idx.reshape(NW, TNB, TKB)
    rr3 = rel_idx.reshape(NW, TNB, TKB)
    tt3 = tail_idx.reshape(NW, TNB, TKB)
    idx3 = jnp.concatenate([hh3, rr3, tt3], axis=1)

    degp = _sc_degree(dst3, ones, zeros)

    g1 = _tc_g1(x, W1, degp)
    acc1 = _sc_edge_scatter(g1, src4, dst4, zeros)
    g2 = _tc_mid(acc1, g1, degp, b1.reshape(1, D), W2)
    acc2 = _sc_edge_scatter(g2, src4, dst4, zeros)
    h = _tc_post2(acc2, g2, degp, b2.reshape(1, D))

    hh, hr, ht = _sc_triple_gather(h, rel_emb, idx3)

    Wm2p = jnp.zeros((D, D), jnp.float32).at[:, :3].set(Wm2)
    bm2p = jnp.zeros((1, D), jnp.float32).at[0, :3].set(bm2)
    out = _tc_mlp(hh, hr, ht, Wm1, bm1.reshape(1, D), Wm2p, bm2p)
    return out[:, :3]


# speedup vs baseline: 1.0553x; 1.0553x over previous
import functools

import jax
import jax.numpy as jnp
from jax import lax
from jax.experimental import pallas as pl
from jax.experimental.pallas import tpu as pltpu
from jax.experimental.pallas import tpu_sc as plsc

NC = 2
NS = 16
NW = NC * NS

N = 10000
E = 320000
D = 128
T = 32768
NREL = 100

EPW = E // NW
KB = 80
NB = EPW // KB
NCH = 5
CNB = NB // NCH
S_LO = 624
S_HI = 640

TPW = T // NW
TKB = 128
TNB = TPW // TKB

_HIGH = lax.Precision.HIGHEST

_mesh = plsc.VectorSubcoreMesh(core_axis_name="c", subcore_axis_name="s")


def _stripe_copy(sid, refs_fn):
    @pl.when(sid < NS - 1)
    def _():
        src, dst = refs_fn(pl.ds(sid * S_LO, S_LO))
        pltpu.sync_copy(src, dst)

    @pl.when(sid == NS - 1)
    def _():
        src, dst = refs_fn(pl.ds((NS - 1) * S_LO, S_HI))
        pltpu.sync_copy(src, dst)


def _sc_degree(dst3, ones, zeros16):
    @functools.partial(
        pl.kernel,
        out_type=jax.ShapeDtypeStruct((NC, N, D), jnp.float32),
        mesh=_mesh,
        scratch_types=[
            pltpu.VMEM((NB, KB), jnp.int32),
            pltpu.VMEM((KB, D), jnp.float32),
            pltpu.VMEM_SHARED((N, D), jnp.float32),
            pltpu.SemaphoreType.DMA,
        ],
    )
    def k(dst_hbm, ones_hbm, zero_hbm, out_hbm, idx_v, ones_v, acc_sh, sem):
        cid = lax.axis_index("c")
        sid = lax.axis_index("s")
        wid = sid * NC + cid
        _stripe_copy(sid, lambda s: (zero_hbm.at[s], acc_sh.at[s]))
        pltpu.sync_copy(ones_hbm, ones_v)
        pltpu.sync_copy(dst_hbm.at[wid], idx_v)
        plsc.subcore_barrier()

        @pl.loop(0, NB)
        def _(j):
            pltpu.async_copy(ones_v, acc_sh.at[idx_v.at[j]], sem, add=True)

        @pl.loop(0, NB)
        def _(j):
            pltpu.make_async_copy(ones_v, acc_sh.at[idx_v.at[j]], sem).wait()

        plsc.subcore_barrier()
        _stripe_copy(sid, lambda s: (acc_sh.at[s], out_hbm.at[cid].at[s]))

    return k(dst3, ones, zeros16)


def _sc_edge_scatter(g, src3, dst3, zeros):
    @functools.partial(
        pl.kernel,
        out_type=jax.ShapeDtypeStruct((NC, N, D), jnp.float32),
        mesh=_mesh,
        scratch_types=[
            pltpu.VMEM((CNB, KB), jnp.int32),
            pltpu.VMEM((CNB, KB), jnp.int32),
        ] + [pltpu.VMEM((KB, D), jnp.float32)] * 3 + [
            pltpu.VMEM_SHARED((N, D), jnp.float32),
        ] + [pltpu.SemaphoreType.DMA] * 6,
    )
    def k(g_hbm, src_hbm, dst_hbm, zero_hbm, out_hbm, src_v, dst_v, *rest):
        rows = rest[:3]
        acc_sh = rest[3]
        gsem = rest[4:7]
        ssem = rest[7:]
        cid = lax.axis_index("c")
        sid = lax.axis_index("s")
        wid = sid * NC + cid
        _stripe_copy(sid, lambda s: (zero_hbm.at[s], acc_sh.at[s]))
        plsc.subcore_barrier()

        for c in range(NCH):
            pltpu.sync_copy(src_hbm.at[wid].at[c], src_v)
            pltpu.sync_copy(dst_hbm.at[wid].at[c], dst_v)
            pltpu.async_copy(g_hbm.at[src_v.at[0]], rows[0], gsem[0])
            pltpu.async_copy(g_hbm.at[src_v.at[1]], rows[1], gsem[1])

            @pl.loop(0, CNB // 3)
            def _(i):
                for off in range(3):
                    j = 3 * i + off
                    nb = (off + 2) % 3
                    pltpu.make_async_copy(
                        g_hbm.at[src_v.at[j]], rows[off], gsem[off]).wait()
                    pltpu.async_copy(rows[off], acc_sh.at[dst_v.at[j]],
                                     ssem[off], add=True)

                    @pl.when(j + 2 < CNB)
                    def _():
                        @pl.when(j >= 1)
                        def _():
                            pltpu.make_async_copy(
                                rows[nb], acc_sh.at[dst_v.at[j - 1]],
                                ssem[nb]).wait()

                        pltpu.async_copy(
                            g_hbm.at[src_v.at[j + 2]], rows[nb], gsem[nb])

            jl = CNB - 1
            pltpu.make_async_copy(
                g_hbm.at[src_v.at[jl]], rows[jl % 3], gsem[jl % 3]).wait()
            pltpu.async_copy(rows[jl % 3], acc_sh.at[dst_v.at[jl]],
                             ssem[jl % 3], add=True)
            for j in range(CNB - 3, CNB):
                pltpu.make_async_copy(
                    rows[j % 3], acc_sh.at[dst_v.at[j]], ssem[j % 3]).wait()

        plsc.subcore_barrier()
        _stripe_copy(sid, lambda s: (acc_sh.at[s], out_hbm.at[cid].at[s]))

    return k(g, src3, dst3, zeros)


def _sc_triple_gather(h, rel_emb, idx3):
    row_t = jax.ShapeDtypeStruct((T, D), jnp.float32)

    @functools.partial(
        pl.kernel,
        out_type=(row_t, row_t, row_t),
        mesh=_mesh,
        scratch_types=[
            pltpu.VMEM((3 * TNB, TKB), jnp.int32),
        ] + [pltpu.VMEM((3, TKB, D), jnp.float32)] * 2
          + [pltpu.SemaphoreType.DMA] * 4,
    )
    def k(h_hbm, rel_hbm, idx_hbm, oh_hbm, orel_hbm, ot_hbm, idx_v, *rest):
        bufs = rest[:2]
        gsem = rest[2:4]
        wsem = rest[4:]
        cid = lax.axis_index("c")
        sid = lax.axis_index("s")
        wid = sid * NC + cid
        base = wid * TPW
        pltpu.sync_copy(idx_hbm.at[wid], idx_v)

        def fire(b, j):
            pltpu.async_copy(h_hbm.at[idx_v.at[j]], bufs[b].at[0], gsem[b])
            pltpu.async_copy(rel_hbm.at[idx_v.at[TNB + j]], bufs[b].at[1],
                             gsem[b])
            pltpu.async_copy(h_hbm.at[idx_v.at[2 * TNB + j]], bufs[b].at[2],
                             gsem[b])

        def wait_gathers(b, j):
            pltpu.make_async_copy(h_hbm.at[idx_v.at[j]], bufs[b].at[0],
                                  gsem[b]).wait()
            pltpu.make_async_copy(rel_hbm.at[idx_v.at[TNB + j]],
                                  bufs[b].at[1], gsem[b]).wait()
            pltpu.make_async_copy(h_hbm.at[idx_v.at[2 * TNB + j]],
                                  bufs[b].at[2], gsem[b]).wait()

        def fire_writes(b, j):
            dst = pl.ds(base + j * TKB, TKB)
            pltpu.async_copy(bufs[b].at[0], oh_hbm.at[dst], wsem[b])
            pltpu.async_copy(bufs[b].at[1], orel_hbm.at[dst], wsem[b])
            pltpu.async_copy(bufs[b].at[2], ot_hbm.at[dst], wsem[b])

        def wait_writes(b, j):
            dst = pl.ds(base + j * TKB, TKB)
            pltpu.make_async_copy(bufs[b].at[0], oh_hbm.at[dst],
                                  wsem[b]).wait()
            pltpu.make_async_copy(bufs[b].at[1], orel_hbm.at[dst],
                                  wsem[b]).wait()
            pltpu.make_async_copy(bufs[b].at[2], ot_hbm.at[dst],
                                  wsem[b]).wait()

        fire(0, 0)
        fire(1, 1)
        for j in range(TNB):
            b = j % 2
            wait_gathers(b, j)
            fire_writes(b, j)
            if j + 2 < TNB:
                wait_writes(b, j)
                fire(b, j + 2)
        for j in range(TNB - 2, TNB):
            wait_writes(j % 2, j)

    return k(h, rel_emb, idx3)


def _dinv_from(degp_a, degp_b):
    return lax.rsqrt(degp_a[:, 0:1] + degp_b[:, 0:1] + 1.0)


def _tc_g1(x, W1, degp):
    def body(x_ref, w_ref, deg_ref, o_ref):
        dinv = _dinv_from(deg_ref[0], deg_ref[1])
        h1 = jnp.dot(x_ref[...], w_ref[...],
                     preferred_element_type=jnp.float32, precision=_HIGH)
        o_ref[...] = h1 * dinv

    return pl.pallas_call(
        body, out_shape=jax.ShapeDtypeStruct((N, D), jnp.float32),
    )(x, W1, degp)


def _tc_mid(accp, g1, degp, b1, W2):
    def body(acc_ref, g_ref, deg_ref, b_ref, w_ref, o_ref):
        dinv = _dinv_from(deg_ref[0], deg_ref[1])
        s = acc_ref[0] + acc_ref[1] + g_ref[...]
        a1 = jnp.maximum(s * dinv + b_ref[...], 0.0)
        h2 = jnp.dot(a1, w_ref[...],
                     preferred_element_type=jnp.float32, precision=_HIGH)
        o_ref[...] = h2 * dinv

    return pl.pallas_call(
        body, out_shape=jax.ShapeDtypeStruct((N, D), jnp.float32),
    )(accp, g1, degp, b1, W2)


def _tc_post2(accp, g2, degp, b2):
    def body(acc_ref, g_ref, deg_ref, b_ref, o_ref):
        dinv = _dinv_from(deg_ref[0], deg_ref[1])
        s = acc_ref[0] + acc_ref[1] + g_ref[...]
        o_ref[...] = jnp.maximum(s * dinv + b_ref[...], 0.0)

    return pl.pallas_call(
        body, out_shape=jax.ShapeDtypeStruct((N, D), jnp.float32),
    )(accp, g2, degp, b2)


_TB = 4096


def _tc_mlp(hh, hr, ht, Wm1, bm1, Wm2p, bm2p):
    def body(hh_ref, hr_ref, ht_ref, w1_ref, b1_ref, w2_ref, b2_ref, o_ref):
        t = hh_ref[...] + hr_ref[...] + ht_ref[...]
        q = jnp.maximum(
            jnp.dot(t, w1_ref[...], preferred_element_type=jnp.float32,
                    precision=_HIGH) + b1_ref[...], 0.0)
        o_ref[...] = jnp.dot(q, w2_ref[...],
                             preferred_element_type=jnp.float32,
                             precision=_HIGH) + b2_ref[...]

    row_spec = pl.BlockSpec((_TB, D), lambda i: (i, 0))
    full = pl.BlockSpec((D, D), lambda i: (0, 0))
    vec = pl.BlockSpec((1, D), lambda i: (0, 0))
    return pl.pallas_call(
        body,
        grid=(T // _TB,),
        in_specs=[row_spec, row_spec, row_spec, full, vec, full, vec],
        out_specs=row_spec,
        out_shape=jax.ShapeDtypeStruct((T, D), jnp.float32),
    )(hh, hr, ht, Wm1, bm1, Wm2p, bm2p)


def kernel(x, edge_index, head_idx, tail_idx, rel_idx, W1, b1, W2, b2,
           rel_emb, Wm1, bm1, Wm2, bm2):
    src3 = edge_index[0].reshape(NW, NB, KB)
    dst3 = edge_index[1].reshape(NW, NB, KB)
    src4 = src3.reshape(NW, NCH, CNB, KB)
    dst4 = dst3.reshape(NW, NCH, CNB, KB)

    ones = jnp.ones((KB, D), jnp.float32)
    zeros = jnp.zeros((N, D), jnp.float32)

    hh3 = head_idx.reshape(NW, TNB, TKB)
    rr3 = rel_idx.reshape(NW, TNB, TKB)
    tt3 = tail_idx.reshape(NW, TNB, TKB)
    idx3 = jnp.concatenate([hh3, rr3, tt3], axis=1)

    degp = _sc_degree(dst3, ones, zeros)

    g1 = _tc_g1(x, W1, degp)
    acc1 = _sc_edge_scatter(g1, src4, dst4, zeros)
    g2 = _tc_mid(acc1, g1, degp, b1.reshape(1, D), W2)
    acc2 = _sc_edge_scatter(g2, src4, dst4, zeros)
    h = _tc_post2(acc2, g2, degp, b2.reshape(1, D))

    hh, hr, ht = _sc_triple_gather(h, rel_emb, idx3)

    Wm2p = jnp.zeros((D, D), jnp.float32).at[:, :3].set(Wm2)
    bm2p = jnp.zeros((1, D), jnp.float32).at[0, :3].set(bm2)
    out = _tc_mlp(hh, hr, ht, Wm1, bm1.reshape(1, D), Wm2p, bm2p)
    return out[:, :3]
